# baseline (speedup 1.0000x reference)
import functools

import numpy as np
import jax
import jax.numpy as jnp
from jax import lax
from jax.experimental import pallas as pl
from jax.experimental.pallas import tpu as pltpu
from jax.experimental.pallas import tpu_sc as plsc

D_MODEL = 768
N_VOCAB = 100000
SEQ = 4096
BATCH = 4
N_ROWS = SEQ * BATCH
SCALE = float(np.sqrt(np.float32(D_MODEL)))

NC, NS = 2, 16
NW = NC * NS
B_PER_W = N_ROWS // NW
CHUNK = 16
N_CHUNKS = B_PER_W // CHUNK
POS_PER_CHUNK = CHUNK // BATCH
LANES = 16
N_VEC = D_MODEL // LANES
NBUF = 8
PREFETCH = 4
DRAIN_LAG = NBUF - PREFETCH
GRP = 8


@functools.lru_cache(maxsize=None)
def _pe_tables():
    position = np.arange(0, SEQ, dtype=np.float32)[:, None]
    two_i = np.arange(0, D_MODEL, 2, dtype=np.float32)
    div_term = np.exp(two_i * -(np.log(10000.0) / D_MODEL))
    enc = np.zeros((SEQ, D_MODEL), dtype=np.float32)
    enc[:, 0::2] = np.sin(position * div_term)
    enc[:, 1::2] = np.cos(position * div_term)

    base = enc[0::GRP]
    baseswap = base.reshape(-1, D_MODEL // 2, 2)[:, :, ::-1].reshape(-1, D_MODEL)
    bs = np.stack([base, baseswap], axis=1).copy()

    freq = np.repeat(div_term, 2)
    d = np.arange(GRP, dtype=np.float32)[:, None]
    cos_d = np.cos(d * freq[None, :]).astype(np.float32)
    sign = np.where(np.arange(D_MODEL) % 2 == 0, 1.0, -1.0).astype(np.float32)
    sin_d = (np.sin(d * freq[None, :]) * sign[None, :]).astype(np.float32)
    ab = np.stack([cos_d, sin_d], axis=1).copy()
    return bs, ab


@functools.partial(
    pl.kernel,
    out_type=jax.ShapeDtypeStruct((SEQ, BATCH, D_MODEL), jnp.float32),
    mesh=plsc.VectorSubcoreMesh(
        core_axis_name="c", subcore_axis_name="s", num_cores=NC
    ),
    scratch_types=[
        pltpu.VMEM((N_CHUNKS, CHUNK), jnp.int32),
        pltpu.VMEM((NBUF, CHUNK, D_MODEL), jnp.float32),
        pltpu.VMEM((NBUF, 2, D_MODEL), jnp.float32),
        pltpu.VMEM((GRP, 2, D_MODEL), jnp.float32),
        pltpu.SemaphoreType.DMA,
        pltpu.SemaphoreType.DMA,
        pltpu.SemaphoreType.DMA,
    ],
)
def _emb_pe_kernel(
    x_hbm, w_hbm, bs_hbm, ab_hbm, out_hbm,
    idx_v, emb_v, bsrow_v, ab_v, gsem, psem, osem,
):
    wid = lax.axis_index("s") * NC + lax.axis_index("c")
    pos_base = wid * (B_PER_W // BATCH)
    grp_base = pos_base // GRP
    pltpu.sync_copy(x_hbm.at[wid], idx_v)
    pltpu.sync_copy(ab_hbm, ab_v)

    def issue_fetch(c):
        b = lax.rem(c, NBUF)
        pltpu.async_copy(w_hbm.at[idx_v.at[c]], emb_v.at[b], gsem)
        pltpu.async_copy(
            bs_hbm.at[grp_base + lax.div(c * POS_PER_CHUNK, GRP)],
            bsrow_v.at[b],
            psem,
        )

    def drain_out_chunk():
        for _ in range(POS_PER_CHUNK):
            pltpu.make_async_copy(
                emb_v.at[0, pl.ds(0, BATCH)], out_hbm.at[0], osem
            ).wait()

    for k in range(PREFETCH):
        issue_fetch(k)

    @pl.loop(0, N_CHUNKS)
    def _chunk_loop(c):
        b = lax.rem(c, NBUF)

        @pl.when(c + PREFETCH < N_CHUNKS)
        def _prefetch_next():
            @pl.when(c >= DRAIN_LAG)
            def _free_buf():
                drain_out_chunk()

            issue_fetch(c + PREFETCH)

        pltpu.make_async_copy(w_hbm.at[idx_v.at[c]], emb_v.at[b], gsem).wait()
        pltpu.make_async_copy(bs_hbm.at[0], bsrow_v.at[b], psem).wait()

        d0 = lax.rem(c * POS_PER_CHUNK, GRP)

        @plsc.parallel_loop(0, N_VEC)
        def _fma_pass(j):
            off = j * LANES
            base_vec = bsrow_v[b, 0, pl.ds(off, LANES)]
            swap_vec = bsrow_v[b, 1, pl.ds(off, LANES)]
            for pp in range(POS_PER_CHUNK):
                pe_vec = (
                    base_vec * ab_v[d0 + pp, 0, pl.ds(off, LANES)]
                    + swap_vec * ab_v[d0 + pp, 1, pl.ds(off, LANES)]
                )
                for bb in range(BATCH):
                    r = pp * BATCH + bb
                    emb_v[b, r, pl.ds(off, LANES)] = (
                        emb_v[b, r, pl.ds(off, LANES)] * SCALE + pe_vec
                    )

        s0 = pos_base + c * POS_PER_CHUNK
        for i in range(POS_PER_CHUNK):
            pltpu.async_copy(
                emb_v.at[b, pl.ds(i * BATCH, BATCH)], out_hbm.at[s0 + i], osem
            )

    for _ in range(NBUF):
        drain_out_chunk()


def kernel(x, W):
    xf = x.astype(jnp.int32).reshape(NW, N_CHUNKS, CHUNK)
    bs, ab = _pe_tables()
    return _emb_pe_kernel(xf, W, jnp.asarray(bs), jnp.asarray(ab))


# speedup vs baseline: 1.0162x; 1.0162x over previous
import functools

import numpy as np
import jax
import jax.numpy as jnp
from jax import lax
from jax.experimental import pallas as pl
from jax.experimental.pallas import tpu as pltpu
from jax.experimental.pallas import tpu_sc as plsc

D_MODEL = 768
N_VOCAB = 100000
SEQ = 4096
BATCH = 4
N_ROWS = SEQ * BATCH
SCALE = float(np.sqrt(np.float32(D_MODEL)))

NC, NS = 2, 16
NW = NC * NS
B_PER_W = N_ROWS // NW
CHUNK = 32
N_CHUNKS = B_PER_W // CHUNK
POS_PER_CHUNK = CHUNK // BATCH
LANES = 16
N_VEC = D_MODEL // LANES
NBUF = 4
PREFETCH = 2
DRAIN_LAG = NBUF - PREFETCH
GRP = 8


@functools.lru_cache(maxsize=None)
def _pe_tables():
    position = np.arange(0, SEQ, dtype=np.float32)[:, None]
    two_i = np.arange(0, D_MODEL, 2, dtype=np.float32)
    div_term = np.exp(two_i * -(np.log(10000.0) / D_MODEL))
    enc = np.zeros((SEQ, D_MODEL), dtype=np.float32)
    enc[:, 0::2] = np.sin(position * div_term)
    enc[:, 1::2] = np.cos(position * div_term)

    base = enc[0::GRP]
    baseswap = base.reshape(-1, D_MODEL // 2, 2)[:, :, ::-1].reshape(-1, D_MODEL)
    bs = np.stack([base, baseswap], axis=1).copy()

    freq = np.repeat(div_term, 2)
    d = np.arange(GRP, dtype=np.float32)[:, None]
    cos_d = np.cos(d * freq[None, :]).astype(np.float32)
    sign = np.where(np.arange(D_MODEL) % 2 == 0, 1.0, -1.0).astype(np.float32)
    sin_d = (np.sin(d * freq[None, :]) * sign[None, :]).astype(np.float32)
    ab = np.stack([cos_d, sin_d], axis=1).copy()
    return bs, ab


@functools.partial(
    pl.kernel,
    out_type=jax.ShapeDtypeStruct((SEQ, BATCH, D_MODEL), jnp.float32),
    mesh=plsc.VectorSubcoreMesh(
        core_axis_name="c", subcore_axis_name="s", num_cores=NC
    ),
    scratch_types=[
        pltpu.VMEM((N_CHUNKS, CHUNK), jnp.int32),
        pltpu.VMEM((NBUF, CHUNK, D_MODEL), jnp.float32),
        pltpu.VMEM((NBUF, 2, D_MODEL), jnp.float32),
        pltpu.VMEM((GRP, 2, D_MODEL), jnp.float32),
        pltpu.SemaphoreType.DMA,
        pltpu.SemaphoreType.DMA,
        pltpu.SemaphoreType.DMA,
    ],
)
def _emb_pe_kernel(
    x_hbm, w_hbm, bs_hbm, ab_hbm, out_hbm,
    idx_v, emb_v, bsrow_v, ab_v, gsem, psem, osem,
):
    wid = lax.axis_index("s") * NC + lax.axis_index("c")
    pos_base = wid * (B_PER_W // BATCH)
    grp_base = pos_base // GRP
    pltpu.sync_copy(x_hbm.at[wid], idx_v)
    pltpu.sync_copy(ab_hbm, ab_v)

    def issue_fetch(c):
        b = lax.rem(c, NBUF)
        pltpu.async_copy(w_hbm.at[idx_v.at[c]], emb_v.at[b], gsem)
        pltpu.async_copy(
            bs_hbm.at[grp_base + lax.div(c * POS_PER_CHUNK, GRP)],
            bsrow_v.at[b],
            psem,
        )

    def drain_out_chunk():
        for _ in range(POS_PER_CHUNK):
            pltpu.make_async_copy(
                emb_v.at[0, pl.ds(0, BATCH)], out_hbm.at[0], osem
            ).wait()

    for k in range(PREFETCH):
        issue_fetch(k)

    @pl.loop(0, N_CHUNKS)
    def _chunk_loop(c):
        b = lax.rem(c, NBUF)

        @pl.when(c + PREFETCH < N_CHUNKS)
        def _prefetch_next():
            @pl.when(c >= DRAIN_LAG)
            def _free_buf():
                drain_out_chunk()

            issue_fetch(c + PREFETCH)

        pltpu.make_async_copy(w_hbm.at[idx_v.at[c]], emb_v.at[b], gsem).wait()
        pltpu.make_async_copy(bs_hbm.at[0], bsrow_v.at[b], psem).wait()

        d0 = lax.rem(c * POS_PER_CHUNK, GRP)

        @plsc.parallel_loop(0, N_VEC)
        def _fma_pass(j):
            off = j * LANES
            base_vec = bsrow_v[b, 0, pl.ds(off, LANES)]
            swap_vec = bsrow_v[b, 1, pl.ds(off, LANES)]
            for pp in range(POS_PER_CHUNK):
                pe_vec = (
                    base_vec * ab_v[d0 + pp, 0, pl.ds(off, LANES)]
                    + swap_vec * ab_v[d0 + pp, 1, pl.ds(off, LANES)]
                )
                for bb in range(BATCH):
                    r = pp * BATCH + bb
                    emb_v[b, r, pl.ds(off, LANES)] = (
                        emb_v[b, r, pl.ds(off, LANES)] * SCALE + pe_vec
                    )

        s0 = pos_base + c * POS_PER_CHUNK
        for i in range(POS_PER_CHUNK):
            pltpu.async_copy(
                emb_v.at[b, pl.ds(i * BATCH, BATCH)], out_hbm.at[s0 + i], osem
            )

    for _ in range(NBUF):
        drain_out_chunk()


def kernel(x, W):
    xf = x.astype(jnp.int32).reshape(NW, N_CHUNKS, CHUNK)
    bs, ab = _pe_tables()
    return _emb_pe_kernel(xf, W, jnp.asarray(bs), jnp.asarray(ab))
